# baseline (speedup 1.0000x reference)
import functools

import jax
import jax.numpy as jnp
from jax import lax
from jax.experimental import pallas as pl
from jax.experimental.pallas import tpu as pltpu
from jax.experimental.pallas import tpu_sc as plsc

K = 16
BIG = 1 << 30


def _norms_body(a_ref, b_ref, out_ref):
    a = a_ref[...]
    b = b_ref[...]
    out_ref[...] = jnp.sum(a * a, axis=-1) + jnp.sum(b * b, axis=-1)


def _norms(patch_tokens, blk=8):
    N, P, D = patch_tokens.shape
    h = D // 2
    return pl.pallas_call(
        _norms_body,
        grid=(N // blk,),
        in_specs=[
            pl.BlockSpec((blk, P, h), lambda i: (i, 0, 0)),
            pl.BlockSpec((blk, P, h), lambda i: (i, 0, 1)),
        ],
        out_specs=pl.BlockSpec((blk, P), lambda i: (i, 0)),
        out_shape=jax.ShapeDtypeStruct((N, P), jnp.float32),
    )(patch_tokens, patch_tokens)


def _topk_body(s_ref, idx_ref, *, N, P):
    s = s_ref[...]
    iota_p = lax.broadcasted_iota(jnp.int32, (N, P), 1)
    vals = s
    cols = []
    for _ in range(K):
        m = jnp.max(vals, axis=1, keepdims=True)
        idx = jnp.min(jnp.where(vals == m, iota_p, BIG), axis=1, keepdims=True)
        cols.append(idx)
        vals = jnp.where(iota_p == idx, -1.0, vals)
    patch_idx = jnp.concatenate(cols, axis=1)
    row_k = lax.broadcasted_iota(jnp.int32, (N, K), 0)
    idx_ref[...] = row_k * P + patch_idx


def _topk_indices(norms):
    N, P = norms.shape
    return pl.pallas_call(
        functools.partial(_topk_body, N=N, P=P),
        out_shape=jax.ShapeDtypeStruct((N, K), jnp.int32),
    )(norms)


def _make_sc_gather(V, D, B):
    nw = 32
    b_per_w = B // nw
    mesh = plsc.VectorSubcoreMesh(core_axis_name="c", subcore_axis_name="s")

    @functools.partial(
        pl.kernel,
        mesh=mesh,
        out_type=jax.ShapeDtypeStruct((B, D), jnp.float32),
        scratch_types=[
            pltpu.VMEM((b_per_w,), jnp.int32),
            pltpu.VMEM((b_per_w, D), jnp.float32),
            pltpu.SemaphoreType.DMA,
        ],
    )
    def gather_kernel(idx_hbm, table_hbm, out_hbm, idx_v, rows_v, sem):
        wid = lax.axis_index("s") * 2 + lax.axis_index("c")
        base = wid * b_per_w
        pltpu.sync_copy(idx_hbm.at[pl.ds(base, b_per_w)], idx_v)
        pltpu.async_copy(table_hbm.at[idx_v], rows_v, sem).wait()
        pltpu.sync_copy(rows_v, out_hbm.at[pl.ds(base, b_per_w)])

    return gather_kernel


def _proj_body(sel_ref, cls_ref, wp_ref, wf_ref, out_ref):
    sel = sel_ref[...]
    wp = wp_ref[...]
    proj = lax.dot_general(
        sel, wp, (((1,), (1,)), ((), ())), preferred_element_type=jnp.float32
    )
    O = proj.shape[-1]
    B = proj.shape[0] // K
    local = jnp.max(proj.reshape(B, K, O), axis=1)

    cls = cls_ref[...]
    wf = wf_ref[...]
    fused = lax.dot_general(
        cls, wf[:, :O], (((1,), (1,)), ((), ())), preferred_element_type=jnp.float32
    ) + lax.dot_general(
        local, wf[:, O:], (((1,), (1,)), ((), ())), preferred_element_type=jnp.float32
    )
    n = jnp.sqrt(jnp.sum(fused * fused, axis=-1, keepdims=True))
    out_ref[...] = fused / jnp.maximum(n, 1e-12)


def _project(selected, cls_feat, W_patch, W_fusion):
    NK, D = selected.shape
    N = NK // K
    O = W_patch.shape[0]
    blk = 32
    return pl.pallas_call(
        _proj_body,
        grid=(N // blk,),
        in_specs=[
            pl.BlockSpec((blk * K, D), lambda i: (i, 0)),
            pl.BlockSpec((blk, O), lambda i: (i, 0)),
            pl.BlockSpec((O, D), lambda i: (0, 0)),
            pl.BlockSpec((O, 2 * O), lambda i: (0, 0)),
        ],
        out_specs=pl.BlockSpec((blk, O), lambda i: (i, 0)),
        out_shape=jax.ShapeDtypeStruct((N, O), jnp.float32),
    )(selected, cls_feat, W_patch, W_fusion)


def kernel(cls_feat, patch_tokens, W_patch, W_fusion):
    N, P, D = patch_tokens.shape
    norms = _norms(patch_tokens)
    idx = _topk_indices(norms)
    table = patch_tokens.reshape(N * P, D)
    selected = _make_sc_gather(N * P, D, N * K)(idx.reshape(N * K), table)
    return _project(selected, cls_feat, W_patch, W_fusion)


# speedup vs baseline: 1.0158x; 1.0158x over previous
import functools

import jax
import jax.numpy as jnp
from jax import lax
from jax.experimental import pallas as pl
from jax.experimental.pallas import tpu as pltpu
from jax.experimental.pallas import tpu_sc as plsc

K = 16
BIG = 1 << 30


def _norms_topk_body(pt_ref, idx_ref, s_ref, *, blk, N, P):
    i = pl.program_id(0)
    pt = pt_ref[...]
    s_ref[pl.ds(i * blk, blk), :] = jnp.sum(pt * pt, axis=-1)

    @pl.when(i == (N // blk) - 1)
    def _():
        s = s_ref[...]
        iota_p = lax.broadcasted_iota(jnp.int32, (N, P), 1)
        vals = s
        cols = []
        for _ in range(K):
            m = jnp.max(vals, axis=1, keepdims=True)
            idx = jnp.min(jnp.where(vals == m, iota_p, BIG), axis=1, keepdims=True)
            cols.append(idx)
            vals = jnp.where(iota_p == idx, -1.0, vals)
        patch_idx = jnp.concatenate(cols, axis=1)
        row_k = lax.broadcasted_iota(jnp.int32, (N, K), 0)
        idx_ref[...] = row_k * P + patch_idx


def _topk_indices(patch_tokens, blk=8):
    N, P, D = patch_tokens.shape
    return pl.pallas_call(
        functools.partial(_norms_topk_body, blk=blk, N=N, P=P),
        grid=(N // blk,),
        in_specs=[pl.BlockSpec((blk, P, D), lambda i: (i, 0, 0))],
        out_specs=pl.BlockSpec((N, K), lambda i: (0, 0)),
        out_shape=jax.ShapeDtypeStruct((N, K), jnp.int32),
        scratch_shapes=[pltpu.VMEM((N, P), jnp.float32)],
    )(patch_tokens)


def _make_sc_gather(V, D, B):
    nw = 32
    b_per_w = B // nw
    mesh = plsc.VectorSubcoreMesh(core_axis_name="c", subcore_axis_name="s")

    @functools.partial(
        pl.kernel,
        mesh=mesh,
        out_type=jax.ShapeDtypeStruct((B, D), jnp.float32),
        scratch_types=[
            pltpu.VMEM((b_per_w,), jnp.int32),
            pltpu.VMEM((b_per_w, D), jnp.float32),
            pltpu.SemaphoreType.DMA,
        ],
    )
    def gather_kernel(idx_hbm, table_hbm, out_hbm, idx_v, rows_v, sem):
        wid = lax.axis_index("s") * 2 + lax.axis_index("c")
        base = wid * b_per_w
        pltpu.sync_copy(idx_hbm.at[pl.ds(base, b_per_w)], idx_v)
        pltpu.async_copy(table_hbm.at[idx_v], rows_v, sem).wait()
        pltpu.sync_copy(rows_v, out_hbm.at[pl.ds(base, b_per_w)])

    return gather_kernel


def _proj_body(sel_ref, cls_ref, wp_ref, wf_ref, out_ref):
    sel = sel_ref[...]
    wp = wp_ref[...]
    proj = lax.dot_general(
        sel, wp, (((1,), (1,)), ((), ())), preferred_element_type=jnp.float32
    )
    O = proj.shape[-1]
    B = proj.shape[0] // K
    local = jnp.max(proj.reshape(B, K, O), axis=1)

    cls = cls_ref[...]
    wf = wf_ref[...]
    fused = lax.dot_general(
        cls, wf[:, :O], (((1,), (1,)), ((), ())), preferred_element_type=jnp.float32
    ) + lax.dot_general(
        local, wf[:, O:], (((1,), (1,)), ((), ())), preferred_element_type=jnp.float32
    )
    n = jnp.sqrt(jnp.sum(fused * fused, axis=-1, keepdims=True))
    out_ref[...] = fused / jnp.maximum(n, 1e-12)


def _project(selected, cls_feat, W_patch, W_fusion):
    NK, D = selected.shape
    N = NK // K
    O = W_patch.shape[0]
    blk = 32
    return pl.pallas_call(
        _proj_body,
        grid=(N // blk,),
        in_specs=[
            pl.BlockSpec((blk * K, D), lambda i: (i, 0)),
            pl.BlockSpec((blk, O), lambda i: (i, 0)),
            pl.BlockSpec((O, D), lambda i: (0, 0)),
            pl.BlockSpec((O, 2 * O), lambda i: (0, 0)),
        ],
        out_specs=pl.BlockSpec((blk, O), lambda i: (i, 0)),
        out_shape=jax.ShapeDtypeStruct((N, O), jnp.float32),
    )(selected, cls_feat, W_patch, W_fusion)


def kernel(cls_feat, patch_tokens, W_patch, W_fusion):
    N, P, D = patch_tokens.shape
    idx = _topk_indices(patch_tokens)
    table = patch_tokens.reshape(N * P, D)
    selected = _make_sc_gather(N * P, D, N * K)(idx.reshape(N * K), table)
    return _project(selected, cls_feat, W_patch, W_fusion)


# speedup vs baseline: 1.0408x; 1.0247x over previous
import functools

import jax
import jax.numpy as jnp
from jax import lax
from jax.experimental import pallas as pl
from jax.experimental.pallas import tpu as pltpu
from jax.experimental.pallas import tpu_sc as plsc

K = 16
BIG = 1 << 30


def _norms_topk_body(pt_ref, idx_ref, s_ref, *, blk, N, P):
    i = pl.program_id(0)
    pt = pt_ref[...]
    s_ref[pl.ds(i * blk, blk), :] = jnp.sum(pt * pt, axis=-1)

    @pl.when(i == (N // blk) - 1)
    def _():
        s = s_ref[...]
        iota_p = lax.broadcasted_iota(jnp.int32, (N, P), 1)
        vals = s
        cols = []
        for _ in range(K):
            m = jnp.max(vals, axis=1, keepdims=True)
            idx = jnp.min(jnp.where(vals == m, iota_p, BIG), axis=1, keepdims=True)
            cols.append(idx)
            vals = jnp.where(iota_p == idx, -1.0, vals)
        patch_idx = jnp.concatenate(cols, axis=1)
        row_k = lax.broadcasted_iota(jnp.int32, (N, K), 0)
        idx_ref[...] = row_k * P + patch_idx


def _topk_indices(patch_tokens, blk=8):
    N, P, D = patch_tokens.shape
    return pl.pallas_call(
        functools.partial(_norms_topk_body, blk=blk, N=N, P=P),
        grid=(N // blk,),
        in_specs=[pl.BlockSpec((blk, P, D), lambda i: (i, 0, 0))],
        out_specs=pl.BlockSpec((N, K), lambda i: (0, 0)),
        out_shape=jax.ShapeDtypeStruct((N, K), jnp.int32),
        scratch_shapes=[pltpu.VMEM((N, P), jnp.float32)],
    )(patch_tokens)


def _make_sc_gather(V, D, N):
    nw = 32
    r_per_w = N // nw
    mesh = plsc.VectorSubcoreMesh(core_axis_name="c", subcore_axis_name="s")

    @functools.partial(
        pl.kernel,
        mesh=mesh,
        out_type=jax.ShapeDtypeStruct((N * K, D), jnp.float32),
        scratch_types=[
            pltpu.VMEM((r_per_w, K), jnp.int32),
            pltpu.VMEM((r_per_w * K, D), jnp.float32),
            pltpu.SemaphoreType.DMA,
        ],
    )
    def gather_kernel(idx_hbm, table_hbm, out_hbm, idx_v, rows_v, sem):
        wid = lax.axis_index("s") * 2 + lax.axis_index("c")
        base = wid * r_per_w
        pltpu.sync_copy(idx_hbm.at[pl.ds(base, r_per_w)], idx_v)
        copies = []
        for j in range(r_per_w):
            vec = idx_v[j]
            copies.append(
                pltpu.async_copy(
                    table_hbm.at[vec], rows_v.at[pl.ds(j * K, K)], sem
                )
            )
        for c in copies:
            c.wait()
        pltpu.sync_copy(rows_v, out_hbm.at[pl.ds(base * K, r_per_w * K)])

    return gather_kernel


def _proj_body(sel_ref, cls_ref, wp_ref, wf_ref, out_ref):
    sel = sel_ref[...]
    wp = wp_ref[...]
    proj = lax.dot_general(
        sel, wp, (((1,), (1,)), ((), ())), preferred_element_type=jnp.float32
    )
    O = proj.shape[-1]
    B = proj.shape[0] // K
    local = jnp.max(proj.reshape(B, K, O), axis=1)

    cls = cls_ref[...]
    wf = wf_ref[...]
    fused = lax.dot_general(
        cls, wf[:, :O], (((1,), (1,)), ((), ())), preferred_element_type=jnp.float32
    ) + lax.dot_general(
        local, wf[:, O:], (((1,), (1,)), ((), ())), preferred_element_type=jnp.float32
    )
    n = jnp.sqrt(jnp.sum(fused * fused, axis=-1, keepdims=True))
    out_ref[...] = fused / jnp.maximum(n, 1e-12)


def _project(selected, cls_feat, W_patch, W_fusion):
    NK, D = selected.shape
    N = NK // K
    O = W_patch.shape[0]
    blk = 32
    return pl.pallas_call(
        _proj_body,
        grid=(N // blk,),
        in_specs=[
            pl.BlockSpec((blk * K, D), lambda i: (i, 0)),
            pl.BlockSpec((blk, O), lambda i: (i, 0)),
            pl.BlockSpec((O, D), lambda i: (0, 0)),
            pl.BlockSpec((O, 2 * O), lambda i: (0, 0)),
        ],
        out_specs=pl.BlockSpec((blk, O), lambda i: (i, 0)),
        out_shape=jax.ShapeDtypeStruct((N, O), jnp.float32),
    )(selected, cls_feat, W_patch, W_fusion)


def kernel(cls_feat, patch_tokens, W_patch, W_fusion):
    N, P, D = patch_tokens.shape
    idx = _topk_indices(patch_tokens)
    table = patch_tokens.reshape(N * P, D)
    selected = _make_sc_gather(N * P, D, N)(idx, table)
    return _project(selected, cls_feat, W_patch, W_fusion)


# speedup vs baseline: 1.0454x; 1.0044x over previous
import functools

import jax
import jax.numpy as jnp
from jax import lax
from jax.experimental import pallas as pl
from jax.experimental.pallas import tpu as pltpu
from jax.experimental.pallas import tpu_sc as plsc

K = 16
BIG = 1 << 30


def _norms_topk_body(pt_ref, idx_ref, s_ref, *, blk, N, P):
    i = pl.program_id(0)
    pt = pt_ref[...]
    s_ref[pl.ds(i * blk, blk), :] = jnp.sum(pt * pt, axis=-1)

    @pl.when(i == (N // blk) - 1)
    def _():
        s = s_ref[...]
        iota_p = lax.broadcasted_iota(jnp.int32, (N, P), 1)
        vals = s
        cols = []
        for _ in range(K):
            m = jnp.max(vals, axis=1, keepdims=True)
            idx = jnp.min(jnp.where(vals == m, iota_p, BIG), axis=1, keepdims=True)
            cols.append(idx)
            vals = jnp.where(iota_p == idx, -1.0, vals)
        patch_idx = jnp.concatenate(cols, axis=1)
        row_k = lax.broadcasted_iota(jnp.int32, (N, K), 0)
        idx_ref[...] = row_k * P + patch_idx


def _topk_indices(patch_tokens, blk=8):
    N, P, D = patch_tokens.shape
    return pl.pallas_call(
        functools.partial(_norms_topk_body, blk=blk, N=N, P=P),
        grid=(N // blk,),
        in_specs=[pl.BlockSpec((blk, P, D), lambda i: (i, 0, 0))],
        out_specs=pl.BlockSpec((N, K), lambda i: (0, 0)),
        out_shape=jax.ShapeDtypeStruct((N, K), jnp.int32),
        scratch_shapes=[pltpu.VMEM((N, P), jnp.float32)],
    )(patch_tokens)


def _make_sc_gather(V, D, N):
    nw = 32
    r_per_w = N // nw
    mesh = plsc.VectorSubcoreMesh(core_axis_name="c", subcore_axis_name="s")

    @functools.partial(
        pl.kernel,
        mesh=mesh,
        out_type=jax.ShapeDtypeStruct((N * K, D), jnp.float32),
        scratch_types=[
            pltpu.VMEM((r_per_w, K), jnp.int32),
            pltpu.VMEM((r_per_w * K, D), jnp.float32),
            pltpu.SemaphoreType.DMA,
        ],
    )
    def gather_kernel(idx_hbm, table_hbm, out_hbm, idx_v, rows_v, sem):
        wid = lax.axis_index("s") * 2 + lax.axis_index("c")
        base = wid * r_per_w
        pltpu.sync_copy(idx_hbm.at[pl.ds(base, r_per_w)], idx_v)
        copies = []
        for j in range(r_per_w):
            vec = idx_v[j]
            copies.append(
                pltpu.async_copy(
                    table_hbm.at[vec], rows_v.at[pl.ds(j * K, K)], sem
                )
            )
        for c in copies:
            c.wait()
        pltpu.sync_copy(rows_v, out_hbm.at[pl.ds(base * K, r_per_w * K)])

    return gather_kernel


def _proj_body(sel_ref, cls_ref, wp_ref, wf_ref, out_ref):
    sel = sel_ref[...]
    wp = wp_ref[...]
    proj = lax.dot_general(
        sel, wp, (((1,), (1,)), ((), ())), preferred_element_type=jnp.float32
    )
    O = proj.shape[-1]
    B = proj.shape[0] // K
    local = jnp.max(proj.reshape(B, K, O), axis=1)

    cls = cls_ref[...]
    wf = wf_ref[...]
    fused = lax.dot_general(
        cls, wf[:, :O], (((1,), (1,)), ((), ())), preferred_element_type=jnp.float32
    ) + lax.dot_general(
        local, wf[:, O:], (((1,), (1,)), ((), ())), preferred_element_type=jnp.float32
    )
    n = jnp.sqrt(jnp.sum(fused * fused, axis=-1, keepdims=True))
    out_ref[...] = fused / jnp.maximum(n, 1e-12)


def _project(selected, cls_feat, W_patch, W_fusion):
    NK, D = selected.shape
    N = NK // K
    O = W_patch.shape[0]
    blk = 64
    return pl.pallas_call(
        _proj_body,
        grid=(N // blk,),
        in_specs=[
            pl.BlockSpec((blk * K, D), lambda i: (i, 0)),
            pl.BlockSpec((blk, O), lambda i: (i, 0)),
            pl.BlockSpec((O, D), lambda i: (0, 0)),
            pl.BlockSpec((O, 2 * O), lambda i: (0, 0)),
        ],
        out_specs=pl.BlockSpec((blk, O), lambda i: (i, 0)),
        out_shape=jax.ShapeDtypeStruct((N, O), jnp.float32),
    )(selected, cls_feat, W_patch, W_fusion)


def kernel(cls_feat, patch_tokens, W_patch, W_fusion):
    N, P, D = patch_tokens.shape
    idx = _topk_indices(patch_tokens)
    table = patch_tokens.reshape(N * P, D)
    selected = _make_sc_gather(N * P, D, N)(idx, table)
    return _project(selected, cls_feat, W_patch, W_fusion)
